# Initial kernel scaffold; baseline (speedup 1.0000x reference)
import jax
import jax.numpy as jnp
from jax.experimental import pallas as pl


def kernel(enc_inputs, sequence_length, current_input, embedding, W_ih, W_hh, b_ih, b_hh):
    raise NotImplementedError("write your pallas kernel here")


# speedup vs baseline: 10.8593x; 10.8593x over previous
import functools

import jax
import jax.numpy as jnp
from jax import lax
from jax.experimental import pallas as pl
from jax.experimental.pallas import tpu as pltpu
from jax.experimental.pallas import tpu_sc as plsc

B, L, V, D, H = 16, 512, 32000, 256, 256
TCH = 64
NT = L // TCH


@functools.lru_cache(maxsize=None)
def _make_sc_gather(n_idx, d):
    info = plsc.get_sparse_core_info()
    nw = info.num_cores * info.num_subcores
    per_w = n_idx // nw
    mesh = plsc.VectorSubcoreMesh(core_axis_name="c", subcore_axis_name="s")

    @functools.partial(
        pl.kernel,
        mesh=mesh,
        out_type=jax.ShapeDtypeStruct((n_idx, d), jnp.float32),
        scratch_types=[
            pltpu.VMEM((per_w,), jnp.int32),
            pltpu.VMEM((per_w, d), jnp.float32),
            pltpu.SemaphoreType.DMA,
        ],
    )
    def gather_k(table_hbm, idx_hbm, out_hbm, idx_v, rows_v, sem):
        wid = lax.axis_index("s") * info.num_cores + lax.axis_index("c")
        base = wid * per_w
        pltpu.sync_copy(idx_hbm.at[pl.ds(base, per_w)], idx_v)
        pltpu.async_copy(table_hbm.at[idx_v], rows_v, sem).wait()
        pltpu.sync_copy(rows_v, out_hbm.at[pl.ds(base, per_w)])

    return gather_k


def _gru_body(sl_ref, emb_ref, wih_ref, whh_ref, bih_ref, bhh_ref,
              out_ref, last_ref, gi_ref, h_ref):
    t = pl.program_id(0)

    @pl.when(t == 0)
    def _():
        h_ref[...] = jnp.zeros_like(h_ref)

    gi_ref[...] = (
        jnp.dot(emb_ref[...], wih_ref[...], preferred_element_type=jnp.float32)
        + bih_ref[...]
    )

    whh = whh_ref[...]
    bhh = bhh_ref[...]
    sl = sl_ref[...]

    def step(j, h):
        gi = gi_ref[pl.ds(j * B, B), :]
        gh = jnp.dot(h, whh, preferred_element_type=jnp.float32) + bhh
        r = jax.nn.sigmoid(gi[:, 0:H] + gh[:, 0:H])
        z = jax.nn.sigmoid(gi[:, H:2 * H] + gh[:, H:2 * H])
        n = jnp.tanh(gi[:, 2 * H:3 * H] + r * gh[:, 2 * H:3 * H])
        h_new = (1.0 - z) * n + z * h
        mt = ((t * TCH + j) < sl).astype(jnp.float32)
        out_ref[pl.ds(j, 1), :, :] = (mt * h_new)[None]
        return mt * h_new + (1.0 - mt) * h

    h = lax.fori_loop(0, TCH, step, h_ref[...])
    h_ref[...] = h
    last_ref[...] = h


def _gru_call(sl_b, emb_tm, wih_t, whh_t, bih, bhh, interpret=False):
    return pl.pallas_call(
        _gru_body,
        grid=(NT,),
        in_specs=[
            pl.BlockSpec((B, H), lambda t: (0, 0)),
            pl.BlockSpec((TCH * B, D), lambda t: (t, 0)),
            pl.BlockSpec((D, 3 * H), lambda t: (0, 0)),
            pl.BlockSpec((H, 3 * H), lambda t: (0, 0)),
            pl.BlockSpec((1, 3 * H), lambda t: (0, 0)),
            pl.BlockSpec((1, 3 * H), lambda t: (0, 0)),
        ],
        out_specs=(
            pl.BlockSpec((TCH, B, H), lambda t: (t, 0, 0)),
            pl.BlockSpec((B, H), lambda t: (0, 0)),
        ),
        out_shape=(
            jax.ShapeDtypeStruct((L, B, H), jnp.float32),
            jax.ShapeDtypeStruct((B, H), jnp.float32),
        ),
        scratch_shapes=[
            pltpu.VMEM((TCH * B, 3 * H), jnp.float32),
            pltpu.VMEM((B, H), jnp.float32),
        ],
        interpret=interpret,
    )(sl_b, emb_tm, wih_t, whh_t, bih, bhh)


def kernel(enc_inputs, sequence_length, current_input, embedding,
           W_ih, W_hh, b_ih, b_hh):
    del current_input
    idx_tm = jnp.swapaxes(enc_inputs, 0, 1).reshape(-1).astype(jnp.int32)
    emb_tm = _make_sc_gather(B * L, D)(embedding, idx_tm)
    sl_b = jnp.broadcast_to(
        sequence_length.astype(jnp.int32)[:, None], (B, H))
    out_tm, last = _gru_call(sl_b, emb_tm, W_ih.T, W_hh.T,
                             b_ih[None, :], b_hh[None, :])
    return jnp.swapaxes(out_tm, 0, 1), last


# speedup vs baseline: 11.8139x; 1.0879x over previous
import functools

import jax
import jax.numpy as jnp
from jax import lax
from jax.experimental import pallas as pl
from jax.experimental.pallas import tpu as pltpu
from jax.experimental.pallas import tpu_sc as plsc

B, L, V, D, H = 16, 512, 32000, 256, 256
TCH = 64
NT = L // TCH
UNROLL = 4


@functools.lru_cache(maxsize=None)
def _make_sc_gather(n_idx, d):
    info = plsc.get_sparse_core_info()
    nw = info.num_cores * info.num_subcores
    per_w = n_idx // nw
    mesh = plsc.VectorSubcoreMesh(core_axis_name="c", subcore_axis_name="s")

    @functools.partial(
        pl.kernel,
        mesh=mesh,
        out_type=jax.ShapeDtypeStruct((n_idx, d), jnp.float32),
        scratch_types=[
            pltpu.VMEM((per_w,), jnp.int32),
            pltpu.VMEM((per_w, d), jnp.float32),
            pltpu.SemaphoreType.DMA,
        ],
    )
    def gather_k(table_hbm, idx_hbm, out_hbm, idx_v, rows_v, sem):
        wid = lax.axis_index("s") * info.num_cores + lax.axis_index("c")
        base = wid * per_w
        pltpu.sync_copy(idx_hbm.at[pl.ds(base, per_w)], idx_v)
        pltpu.async_copy(table_hbm.at[idx_v], rows_v, sem).wait()
        pltpu.sync_copy(rows_v, out_hbm.at[pl.ds(base, per_w)])

    return gather_k


def _gru_body(sl_ref, emb_ref, wih_ref, whh_ref, bih_ref, bhh_ref,
              out_ref, last_ref, gi_ref, h_ref):
    t = pl.program_id(0)

    @pl.when(t == 0)
    def _():
        h_ref[...] = jnp.zeros_like(h_ref)

    gi_ref[...] = (
        jnp.dot(emb_ref[...], wih_ref[...], preferred_element_type=jnp.float32)
        + bih_ref[...]
    )

    whh = whh_ref[...]
    bhh = bhh_ref[...]
    sl = sl_ref[...]

    def one_step(j, h):
        gi = gi_ref[pl.ds(j * B, B), :]
        gh = jnp.dot(h.astype(jnp.bfloat16), whh,
                     preferred_element_type=jnp.float32) + bhh
        r = jax.nn.sigmoid(gi[:, 0:H] + gh[:, 0:H])
        z = jax.nn.sigmoid(gi[:, H:2 * H] + gh[:, H:2 * H])
        n = jnp.tanh(gi[:, 2 * H:3 * H] + r * gh[:, 2 * H:3 * H])
        h_new = (1.0 - z) * n + z * h
        mt = ((t * TCH + j) < sl).astype(jnp.float32)
        out_ref[pl.ds(j, 1), :, :] = (mt * h_new)[None]
        return mt * h_new + (1.0 - mt) * h

    def step4(g, h):
        for u in range(UNROLL):
            h = one_step(g * UNROLL + u, h)
        return h

    h = lax.fori_loop(0, TCH // UNROLL, step4, h_ref[...])
    h_ref[...] = h
    last_ref[...] = h


def _gru_call(sl_b, emb_tm, wih_t, whh_t, bih, bhh, interpret=False):
    return pl.pallas_call(
        _gru_body,
        grid=(NT,),
        in_specs=[
            pl.BlockSpec((B, H), lambda t: (0, 0)),
            pl.BlockSpec((TCH * B, D), lambda t: (t, 0)),
            pl.BlockSpec((D, 3 * H), lambda t: (0, 0)),
            pl.BlockSpec((H, 3 * H), lambda t: (0, 0)),
            pl.BlockSpec((1, 3 * H), lambda t: (0, 0)),
            pl.BlockSpec((1, 3 * H), lambda t: (0, 0)),
        ],
        out_specs=(
            pl.BlockSpec((TCH, B, H), lambda t: (t, 0, 0)),
            pl.BlockSpec((B, H), lambda t: (0, 0)),
        ),
        out_shape=(
            jax.ShapeDtypeStruct((L, B, H), jnp.float32),
            jax.ShapeDtypeStruct((B, H), jnp.float32),
        ),
        scratch_shapes=[
            pltpu.VMEM((TCH * B, 3 * H), jnp.float32),
            pltpu.VMEM((B, H), jnp.float32),
        ],
        interpret=interpret,
    )(sl_b, emb_tm, wih_t, whh_t, bih, bhh)


def kernel(enc_inputs, sequence_length, current_input, embedding,
           W_ih, W_hh, b_ih, b_hh):
    del current_input
    idx_tm = jnp.swapaxes(enc_inputs, 0, 1).reshape(-1).astype(jnp.int32)
    emb_tm = _make_sc_gather(B * L, D)(embedding, idx_tm)
    sl_b = jnp.broadcast_to(
        sequence_length.astype(jnp.int32)[:, None], (B, H))
    out_tm, last = _gru_call(sl_b, emb_tm, W_ih.T,
                             W_hh.T.astype(jnp.bfloat16),
                             b_ih[None, :], b_hh[None, :])
    return jnp.swapaxes(out_tm, 0, 1), last
